# Initial kernel scaffold; baseline (speedup 1.0000x reference)
import jax
import jax.numpy as jnp
from jax.experimental import pallas as pl


def kernel(x, edge_index, edge_attr, batch, Wemb, bemb, Wq, bq, Wk, bk, Wv, bv, We, Wskip, bskip, Wbeta, g1, bn1, g2, bn2, Wf1, bf1, Wf2, bf2, Wout, bout):
    raise NotImplementedError("write your pallas kernel here")


# speedup vs baseline: 19.4661x; 19.4661x over previous
import functools

import jax
import jax.numpy as jnp
from jax import lax
from jax.experimental import pallas as pl
from jax.experimental.pallas import tpu as pltpu
from jax.experimental.pallas import tpu_sc as plsc

_D = 128
_H = 8
_C = 16
_NG = 64
_NC = 2
_NS = 16
_NW = _NC * _NS
_BLK = 40


def _mm_bias(x, W, b, block_rows):
    n, k = x.shape
    m = W.shape[1]

    def body(x_ref, w_ref, b_ref, o_ref):
        o_ref[...] = (
            jnp.dot(x_ref[...], w_ref[...], preferred_element_type=jnp.float32)
            + b_ref[...]
        )

    return pl.pallas_call(
        body,
        grid=(n // block_rows,),
        in_specs=[
            pl.BlockSpec((block_rows, k), lambda i: (i, 0)),
            pl.BlockSpec((k, m), lambda i: (0, 0)),
            pl.BlockSpec((1, m), lambda i: (0, 0)),
        ],
        out_specs=pl.BlockSpec((block_rows, m), lambda i: (i, 0)),
        out_shape=jax.ShapeDtypeStruct((n, m), jnp.float32),
    )(x, W, b)


def _qkv_skip(h, W4, b4, block_rows):
    n, k = h.shape

    def body(h_ref, w_ref, b_ref, q_ref, kv_ref, xr_ref):
        o = (
            jnp.dot(h_ref[...], w_ref[...], preferred_element_type=jnp.float32)
            + b_ref[...]
        )
        q_ref[...] = o[:, :_D]
        kv_ref[...] = o[:, _D:3 * _D]
        xr_ref[...] = o[:, 3 * _D:]

    return pl.pallas_call(
        body,
        grid=(n // block_rows,),
        in_specs=[
            pl.BlockSpec((block_rows, k), lambda i: (i, 0)),
            pl.BlockSpec((k, 4 * _D), lambda i: (0, 0)),
            pl.BlockSpec((1, 4 * _D), lambda i: (0, 0)),
        ],
        out_specs=[
            pl.BlockSpec((block_rows, _D), lambda i: (i, 0)),
            pl.BlockSpec((block_rows, 2 * _D), lambda i: (i, 0)),
            pl.BlockSpec((block_rows, _D), lambda i: (i, 0)),
        ],
        out_shape=[
            jax.ShapeDtypeStruct((n, _D), jnp.float32),
            jax.ShapeDtypeStruct((n, 2 * _D), jnp.float32),
            jax.ShapeDtypeStruct((n, _D), jnp.float32),
        ],
    )(h, W4, b4)


def _edge_math(qg, kvg, edge_attr, We, scale, block_rows):
    e_total = qg.shape[0]
    ed = edge_attr.shape[1]

    def body(q_ref, kv_ref, ea_ref, we_ref, swvj_ref, s16_ref):
        q = q_ref[...]
        k = kv_ref[:, :_D]
        v = kv_ref[:, _D:]
        e = jnp.dot(ea_ref[...], we_ref[...], preferred_element_type=jnp.float32)
        kj = k + e
        vj = v + e
        p = q * kj
        ii = lax.broadcasted_iota(jnp.int32, (_D, _D), 0)
        jj = lax.broadcasted_iota(jnp.int32, (_D, _D), 1)
        bd = ((ii // _C) == (jj // _C)).astype(jnp.float32)
        ab = jnp.dot(p, bd, preferred_element_type=jnp.float32) * scale
        sb = jnp.exp(ab)
        swvj_ref[...] = sb * vj
        s16_ref[...] = sb

    return pl.pallas_call(
        body,
        grid=(e_total // block_rows,),
        in_specs=[
            pl.BlockSpec((block_rows, _D), lambda i: (i, 0)),
            pl.BlockSpec((block_rows, 2 * _D), lambda i: (i, 0)),
            pl.BlockSpec((block_rows, ed), lambda i: (i, 0)),
            pl.BlockSpec((ed, _D), lambda i: (0, 0)),
        ],
        out_specs=[
            pl.BlockSpec((block_rows, _D), lambda i: (i, 0)),
            pl.BlockSpec((block_rows, _D), lambda i: (i, 0)),
        ],
        out_shape=[
            jax.ShapeDtypeStruct((e_total, _D), jnp.float32),
            jax.ShapeDtypeStruct((e_total, _D), jnp.float32),
        ],
    )(qg, kvg, edge_attr, We)


def _gate_ln(op0, op1, dp0, dp1, xr, h, Av, Bv, g1, b1, block_rows):
    n = h.shape[0]

    def body(o0_ref, o1_ref, d0_ref, d1_ref, xr_ref, h_ref, a_ref, b_ref,
             g_ref, bb_ref, out_ref):
        den = d0_ref[...] + d1_ref[...]
        out = (o0_ref[...] + o1_ref[...]) / jnp.maximum(den, 1e-30)
        xrv = xr_ref[...]
        t = (jnp.sum(out * a_ref[...], axis=1, keepdims=True)
             + jnp.sum(xrv * b_ref[...], axis=1, keepdims=True))
        beta = jax.nn.sigmoid(t)
        out2 = beta * xrv + (1.0 - beta) * out
        y = out2 + h_ref[...]
        m = jnp.mean(y, axis=1, keepdims=True)
        yc = y - m
        var = jnp.mean(yc * yc, axis=1, keepdims=True)
        out_ref[...] = yc * lax.rsqrt(var + 1e-5) * g_ref[...] + bb_ref[...]

    row = lambda i: (i, 0)
    fixed = lambda i: (0, 0)
    return pl.pallas_call(
        body,
        grid=(n // block_rows,),
        in_specs=[
            pl.BlockSpec((block_rows, _D), row),
            pl.BlockSpec((block_rows, _D), row),
            pl.BlockSpec((block_rows, _D), row),
            pl.BlockSpec((block_rows, _D), row),
            pl.BlockSpec((block_rows, _D), row),
            pl.BlockSpec((block_rows, _D), row),
            pl.BlockSpec((1, _D), fixed),
            pl.BlockSpec((1, _D), fixed),
            pl.BlockSpec((1, _D), fixed),
            pl.BlockSpec((1, _D), fixed),
        ],
        out_specs=pl.BlockSpec((block_rows, _D), row),
        out_shape=jax.ShapeDtypeStruct((n, _D), jnp.float32),
    )(op0, op1, dp0, dp1, xr, h, Av, Bv, g1, b1)


def _ffn_ln(h, W1, b1, W2, b2, g2, bn2, block_rows):
    n = h.shape[0]
    dff = W1.shape[1]

    def body(h_ref, w1_ref, b1_ref, w2_ref, b2_ref, g_ref, bb_ref, o_ref):
        hv = h_ref[...]
        f = jnp.maximum(
            jnp.dot(hv, w1_ref[...], preferred_element_type=jnp.float32)
            + b1_ref[...], 0.0)
        f2 = (jnp.dot(f, w2_ref[...], preferred_element_type=jnp.float32)
              + b2_ref[...])
        y = f2 + hv
        m = jnp.mean(y, axis=1, keepdims=True)
        yc = y - m
        var = jnp.mean(yc * yc, axis=1, keepdims=True)
        o_ref[...] = yc * lax.rsqrt(var + 1e-5) * g_ref[...] + bb_ref[...]

    row = lambda i: (i, 0)
    fixed = lambda i: (0, 0)
    return pl.pallas_call(
        body,
        grid=(n // block_rows,),
        in_specs=[
            pl.BlockSpec((block_rows, _D), row),
            pl.BlockSpec((_D, dff), fixed),
            pl.BlockSpec((1, dff), fixed),
            pl.BlockSpec((dff, _D), fixed),
            pl.BlockSpec((1, _D), fixed),
            pl.BlockSpec((1, _D), fixed),
            pl.BlockSpec((1, _D), fixed),
        ],
        out_specs=pl.BlockSpec((block_rows, _D), row),
        out_shape=jax.ShapeDtypeStruct((n, _D), jnp.float32),
    )(h, W1, b1, W2, b2, g2, bn2)


def _pool(h, batch3, block_rows):
    n = h.shape[0]
    nblk = n // block_rows

    def body(b_ref, h_ref, sums_ref, cnt_ref):
        i = pl.program_id(0)

        @pl.when(i == 0)
        def _():
            sums_ref[...] = jnp.zeros_like(sums_ref)
            cnt_ref[...] = jnp.zeros_like(cnt_ref)

        ids = b_ref[0, 0, :]
        gid = lax.broadcasted_iota(jnp.int32, (_NG, block_rows), 0)
        onehot = (gid == ids[None, :]).astype(jnp.float32)
        sums_ref[...] += jnp.dot(onehot, h_ref[...],
                                 preferred_element_type=jnp.float32)
        c = jnp.sum(onehot, axis=1, keepdims=True)
        cnt_ref[...] += jnp.broadcast_to(c, (_NG, _D))

    return pl.pallas_call(
        body,
        grid=(nblk,),
        in_specs=[
            pl.BlockSpec((1, 1, block_rows), lambda i: (i, 0, 0)),
            pl.BlockSpec((block_rows, _D), lambda i: (i, 0)),
        ],
        out_specs=[
            pl.BlockSpec((_NG, _D), lambda i: (0, 0)),
            pl.BlockSpec((_NG, _D), lambda i: (0, 0)),
        ],
        out_shape=[
            jax.ShapeDtypeStruct((_NG, _D), jnp.float32),
            jax.ShapeDtypeStruct((_NG, _D), jnp.float32),
        ],
    )(batch3, h)


def _final(sums, cnt, Wout, bout):
    def body(s_ref, c_ref, w_ref, b_ref, o_ref):
        g = s_ref[...] / jnp.maximum(c_ref[...], 1.0)
        o_ref[...] = (jnp.dot(g, w_ref[...], preferred_element_type=jnp.float32)
                      + b_ref[...])

    return pl.pallas_call(
        body,
        out_shape=jax.ShapeDtypeStruct((_NG, _D), jnp.float32),
    )(sums, cnt, Wout, bout)


def _sc_gather(q, kv, dst, src):
    e_total = dst.shape[0]
    per_w = e_total // _NW
    steps = per_w // _BLK
    mesh = plsc.VectorSubcoreMesh(core_axis_name="c", subcore_axis_name="s")

    @functools.partial(
        pl.kernel, mesh=mesh,
        out_type=[
            jax.ShapeDtypeStruct((e_total, _D), jnp.float32),
            jax.ShapeDtypeStruct((e_total, 2 * _D), jnp.float32),
        ],
        scratch_types=[
            pltpu.VMEM((_BLK,), jnp.int32),
            pltpu.VMEM((_BLK,), jnp.int32),
            pltpu.VMEM((_BLK, _D), jnp.float32),
            pltpu.VMEM((_BLK, 2 * _D), jnp.float32),
        ],
    )
    def k(q_hbm, kv_hbm, dst_hbm, src_hbm, qg_hbm, kvg_hbm,
          dsti, srci, qrows, kvrows):
        wid = lax.axis_index("s") * _NC + lax.axis_index("c")
        base = wid * per_w

        @pl.loop(0, steps)
        def _(j):
            off = base + j * _BLK
            pltpu.sync_copy(dst_hbm.at[pl.ds(off, _BLK)], dsti)
            pltpu.sync_copy(src_hbm.at[pl.ds(off, _BLK)], srci)
            pltpu.sync_copy(q_hbm.at[dsti], qrows)
            pltpu.sync_copy(kv_hbm.at[srci], kvrows)
            pltpu.sync_copy(qrows, qg_hbm.at[pl.ds(off, _BLK)])
            pltpu.sync_copy(kvrows, kvg_hbm.at[pl.ds(off, _BLK)])

    return k(q, kv, dst, src)


def _sc_scatter_add(vals, dst1, zeros, n):
    e_total = vals.shape[0]
    w = vals.shape[1]
    per_w = e_total // _NW
    blk = _BLK
    steps = per_w // blk
    stride = 624
    span = 640
    mesh = plsc.VectorSubcoreMesh(core_axis_name="c", subcore_axis_name="s")

    @functools.partial(
        pl.kernel, mesh=mesh,
        out_type=jax.ShapeDtypeStruct((_NC, n, w), jnp.float32),
        scratch_types=[
            pltpu.VMEM((1, blk), jnp.int32),
            pltpu.VMEM((blk, w), jnp.float32),
            pltpu.VMEM_SHARED((n, w), jnp.float32),
        ],
    )
    def k(vals_hbm, dst_hbm, z_hbm, outp_hbm, dsti2, rows, acc):
        cid = lax.axis_index("c")
        sid = lax.axis_index("s")
        wid = sid * _NC + cid
        rbase = sid * stride

        pltpu.sync_copy(z_hbm.at[pl.ds(rbase, span)],
                        acc.at[pl.ds(rbase, span)])
        plsc.subcore_barrier()
        base = wid * per_w

        @pl.loop(0, steps)
        def _(j):
            off = base + j * blk
            pltpu.sync_copy(dst_hbm.at[pl.ds(off, blk)], dsti2.at[0])
            pltpu.sync_copy(vals_hbm.at[pl.ds(off, blk)], rows)
            pltpu.sync_copy(rows, acc.at[dsti2.at[0]], add=True)

        plsc.subcore_barrier()
        pltpu.sync_copy(acc.at[pl.ds(rbase, span)],
                        outp_hbm.at[cid, pl.ds(rbase, span)])

    return k(vals, dst1, zeros)


def kernel(x, edge_index, edge_attr, batch, Wemb, bemb, Wq, bq, Wk, bk, Wv, bv,
           We, Wskip, bskip, Wbeta, g1, bn1, g2, bn2, Wf1, bf1, Wf2, bf2,
           Wout, bout):
    n = x.shape[0]
    a = x.shape[1]
    e_total = edge_index.shape[1]
    nlayers = Wq.shape[0]
    scale = 1.0 / float(_C) ** 0.5

    src = edge_index[0]
    dst = edge_index[1]
    z128 = jnp.zeros((n, _D), jnp.float32)

    xp = jnp.pad(x, ((0, 0), (0, _D - a)))
    Wembp = jnp.pad(Wemb, ((0, _D - a), (0, 0)))
    h = _mm_bias(xp, Wembp, bemb.reshape(1, _D), 2000)

    for l in range(nlayers):
        W4 = jnp.concatenate([Wq[l], Wk[l], Wv[l], Wskip[l]], axis=1)
        b4 = jnp.concatenate([bq[l], bk[l], bv[l], bskip[l]]).reshape(1, 4 * _D)
        q, kv, xr = _qkv_skip(h, W4, b4, 2000)
        qg, kvg = _sc_gather(q, kv, dst, src)
        swvj, s16 = _edge_math(qg, kvg, edge_attr, We[l], scale, 2000)
        outp = _sc_scatter_add(swvj, dst, z128, n)
        denp = _sc_scatter_add(s16, dst, z128 + outp[0, 0, 0] * 0.0, n)
        Av = (Wbeta[l, :_D, 0] + Wbeta[l, 2 * _D:, 0]).reshape(1, _D)
        Bv = (Wbeta[l, _D:2 * _D, 0] - Wbeta[l, 2 * _D:, 0]).reshape(1, _D)
        h = _gate_ln(outp[0], outp[1], denp[0], denp[1], xr, h, Av, Bv,
                     g1[l].reshape(1, _D), bn1[l].reshape(1, _D), 2000)
        h = _ffn_ln(h, Wf1[l], bf1[l].reshape(1, 4 * _D), Wf2[l],
                    bf2[l].reshape(1, _D), g2[l].reshape(1, _D),
                    bn2[l].reshape(1, _D), 2000)

    batch3 = batch.reshape(n // 2000, 1, 2000)
    sums, cnt = _pool(h, batch3, 2000)
    return _final(sums, cnt, Wout, bout.reshape(1, _D))
